# baseline (speedup 1.0000x reference)
import jax
import jax.numpy as jnp
from jax import lax
from jax.experimental import pallas as pl
from jax.experimental.pallas import tpu as pltpu

NUM_LAYERS = 2
POOL_SIZE = 1000
LENGTH = 5
NUM_HEADS = 12
EMBED_DIM = 768
HEAD_DIM = EMBED_DIM // NUM_HEADS
TOP_K = 4
BATCH = 128
SEQ = 197
SEQ_WIN = 66

XB = 8
NXB = BATCH // XB
HB = 4
NHB = NUM_HEADS // HB


def _seqsum_body(x_ref, o_ref):
    def _win(lo, hi):
        def _acc(si, acc):
            return acc + x_ref[si, :, :]
        return lax.fori_loop(lo, hi, _acc,
                             jnp.zeros((XB, EMBED_DIM), jnp.float32))

    o_ref[...] = (_win(0, SEQ_WIN) + _win(SEQ_WIN, 2 * SEQ_WIN)) + _win(
        2 * SEQ_WIN, SEQ)


def _topk_gather_body(s_ref, invx_ref, keyn_ref, p_ref, rs_ref, o_ref, oh_ref):
    i = pl.program_id(0)

    @pl.when(i == 0)
    def _topk():
        xn = s_ref[...] * invx_ref[...]
        keyn = keyn_ref[...]
        sim = lax.dot_general(xn, keyn, (((1,), (1,)), ((), ())),
                              preferred_element_type=jnp.float32)
        iota = lax.broadcasted_iota(jnp.int32, sim.shape, 1)
        total = jnp.float32(0.0)
        masks = []
        for _ in range(TOP_K):
            m = jnp.max(sim, axis=1, keepdims=True)
            am = jnp.min(jnp.where(sim == m, iota, jnp.int32(2**30)),
                         axis=1, keepdims=True)
            hit = iota == am
            masks.append(hit.astype(jnp.float32))
            total = total + jnp.sum(m)
            sim = jnp.where(hit, -jnp.inf, sim)
        rs_ref[...] = jnp.full((1, 1), total / BATCH, jnp.float32)

        bi = lax.broadcasted_iota(jnp.int32, (BATCH, 64), 0)
        co = lax.broadcasted_iota(jnp.int32, (BATCH, 64), 1)
        dn = (((0,), (0,)), ((), ()))
        for dp in range(2):
            t = (bi == 2 * co + dp).astype(jnp.float32)
            for k in range(TOP_K):
                j = dp * TOP_K + k
                oh_ref[:, pl.ds(j * 64, 64)] = lax.dot_general(
                    masks[k], t, dn, preferred_element_type=jnp.float32)

    @pl.when(i > 0)
    def _gather_mm():
        x = p_ref[...]
        a = x.reshape(2, LENGTH * HB * HEAD_DIM, POOL_SIZE)
        dn = (((1,), (0,)), ((), ()))
        oh = oh_ref[...]
        for d in range(2):
            out = lax.dot_general(a[d], oh, dn,
                                  preferred_element_type=jnp.float32)
            for j in range(2 * TOP_K):
                o_ref[0, j, :, :, :, pl.ds(d * 64, 64)] = out[
                    :, j * 64:(j + 1) * 64].reshape(LENGTH, HB, HEAD_DIM, 64)


def kernel(x_embed, prompt, prompt_key):
    xt = jnp.transpose(x_embed, (1, 0, 2))
    ssum = pl.pallas_call(
        _seqsum_body,
        grid=(NXB,),
        in_specs=[pl.BlockSpec((SEQ, XB, EMBED_DIM), lambda i: (0, i, 0))],
        out_specs=pl.BlockSpec((XB, EMBED_DIM), lambda i: (i, 0)),
        out_shape=jax.ShapeDtypeStruct((BATCH, EMBED_DIM), jnp.float32),
    )(xt)

    c = jnp.float32(1.0) / jnp.float32(197.0)
    mv = ssum * c
    ssq = jnp.sum(jnp.square(mv), axis=-1, keepdims=True)
    inv_x = lax.rsqrt(jnp.maximum(ssq, 1e-12)) * c
    ssk = jnp.sum(jnp.square(prompt_key), axis=-1, keepdims=True)
    keyn = prompt_key * lax.rsqrt(jnp.maximum(ssk, 1e-12))

    pt = jnp.transpose(prompt, (0, 1, 3, 4, 5, 2))

    def _pt_map(i):
        g = jnp.maximum(i - 1, 0)
        return (g // NHB, 0, 0, g % NHB, 0, 0)

    rs, a7 = pl.pallas_call(
        _topk_gather_body,
        grid=(1 + NUM_LAYERS * NHB,),
        in_specs=[
            pl.BlockSpec((BATCH, EMBED_DIM), lambda i: (0, 0)),
            pl.BlockSpec((BATCH, 1), lambda i: (0, 0)),
            pl.BlockSpec((POOL_SIZE, EMBED_DIM), lambda i: (0, 0)),
            pl.BlockSpec((1, 2, LENGTH, HB, HEAD_DIM, POOL_SIZE), _pt_map),
        ],
        out_specs=[
            pl.BlockSpec((1, 1), lambda i: (0, 0)),
            pl.BlockSpec((1, 2 * TOP_K, LENGTH, HB, HEAD_DIM, BATCH), _pt_map),
        ],
        out_shape=[
            jax.ShapeDtypeStruct((1, 1), jnp.float32),
            jax.ShapeDtypeStruct(
                (NUM_LAYERS, 2 * TOP_K, LENGTH, NUM_HEADS, HEAD_DIM, BATCH),
                jnp.float32),
        ],
        scratch_shapes=[pltpu.VMEM((POOL_SIZE, 8 * 64), jnp.float32)],
    )(ssum, inv_x, keyn, pt)

    a6 = a7.reshape(NUM_LAYERS, 2, TOP_K * LENGTH, NUM_HEADS, HEAD_DIM, BATCH)
    batched_prompt = jnp.transpose(a6, (0, 5, 1, 2, 3, 4))
    return (batched_prompt, rs[0, 0])


# speedup vs baseline: 1.0666x; 1.0666x over previous
import jax
import jax.numpy as jnp
from jax import lax
from jax.experimental import pallas as pl
from jax.experimental.pallas import tpu as pltpu

NUM_LAYERS = 2
POOL_SIZE = 1000
LENGTH = 5
NUM_HEADS = 12
EMBED_DIM = 768
HEAD_DIM = EMBED_DIM // NUM_HEADS
TOP_K = 4
BATCH = 128
SEQ = 197
SEQ_WIN = 66

XB = 16
NXB = BATCH // XB
HB = 4
NHB = NUM_HEADS // HB


def _seqsum_body(x_ref, o_ref):
    def _win(lo, hi):
        def _acc(si, acc):
            return acc + x_ref[si, :, :]
        return lax.fori_loop(lo, hi, _acc,
                             jnp.zeros((XB, EMBED_DIM), jnp.float32))

    o_ref[...] = (_win(0, SEQ_WIN) + _win(SEQ_WIN, 2 * SEQ_WIN)) + _win(
        2 * SEQ_WIN, SEQ)


def _topk_gather_body(s_ref, invx_ref, keyn_ref, p_ref, rs_ref, o_ref, oh_ref):
    i = pl.program_id(0)

    @pl.when(i == 0)
    def _topk():
        xn = s_ref[...] * invx_ref[...]
        keyn = keyn_ref[...]
        sim = lax.dot_general(xn, keyn, (((1,), (1,)), ((), ())),
                              preferred_element_type=jnp.float32)
        iota = lax.broadcasted_iota(jnp.int32, sim.shape, 1)
        total = jnp.float32(0.0)
        masks = []
        for _ in range(TOP_K):
            m = jnp.max(sim, axis=1, keepdims=True)
            am = jnp.min(jnp.where(sim == m, iota, jnp.int32(2**30)),
                         axis=1, keepdims=True)
            hit = iota == am
            masks.append(hit.astype(jnp.float32))
            total = total + jnp.sum(m)
            sim = jnp.where(hit, -jnp.inf, sim)
        rs_ref[...] = jnp.full((1, 1), total / BATCH, jnp.float32)

        bi = lax.broadcasted_iota(jnp.int32, (BATCH, 64), 0)
        co = lax.broadcasted_iota(jnp.int32, (BATCH, 64), 1)
        dn = (((0,), (0,)), ((), ()))
        for dp in range(2):
            t = (bi == 2 * co + dp).astype(jnp.float32)
            for k in range(TOP_K):
                j = dp * TOP_K + k
                oh_ref[:, pl.ds(j * 64, 64)] = lax.dot_general(
                    masks[k], t, dn, preferred_element_type=jnp.float32)

    @pl.when(i > 0)
    def _gather_mm():
        x = p_ref[...]
        a = x.reshape(2, LENGTH * HB * HEAD_DIM, POOL_SIZE)
        dn = (((1,), (0,)), ((), ()))
        oh = oh_ref[...]
        for d in range(2):
            out = lax.dot_general(a[d], oh, dn,
                                  preferred_element_type=jnp.float32)
            for j in range(2 * TOP_K):
                o_ref[0, j, :, :, :, pl.ds(d * 64, 64)] = out[
                    :, j * 64:(j + 1) * 64].reshape(LENGTH, HB, HEAD_DIM, 64)


def kernel(x_embed, prompt, prompt_key):
    xt = jnp.transpose(x_embed, (1, 0, 2))
    ssum = pl.pallas_call(
        _seqsum_body,
        grid=(NXB,),
        in_specs=[pl.BlockSpec((SEQ, XB, EMBED_DIM), lambda i: (0, i, 0))],
        out_specs=pl.BlockSpec((XB, EMBED_DIM), lambda i: (i, 0)),
        out_shape=jax.ShapeDtypeStruct((BATCH, EMBED_DIM), jnp.float32),
    )(xt)

    c = jnp.float32(1.0) / jnp.float32(197.0)
    mv = ssum * c
    ssq = jnp.sum(jnp.square(mv), axis=-1, keepdims=True)
    inv_x = lax.rsqrt(jnp.maximum(ssq, 1e-12)) * c
    ssk = jnp.sum(jnp.square(prompt_key), axis=-1, keepdims=True)
    keyn = prompt_key * lax.rsqrt(jnp.maximum(ssk, 1e-12))

    pt = jnp.transpose(prompt, (0, 1, 3, 4, 5, 2))

    def _pt_map(i):
        g = jnp.maximum(i - 1, 0)
        return (g // NHB, 0, 0, g % NHB, 0, 0)

    rs, a7 = pl.pallas_call(
        _topk_gather_body,
        grid=(1 + NUM_LAYERS * NHB,),
        in_specs=[
            pl.BlockSpec((BATCH, EMBED_DIM), lambda i: (0, 0)),
            pl.BlockSpec((BATCH, 1), lambda i: (0, 0)),
            pl.BlockSpec((POOL_SIZE, EMBED_DIM), lambda i: (0, 0)),
            pl.BlockSpec((1, 2, LENGTH, HB, HEAD_DIM, POOL_SIZE), _pt_map),
        ],
        out_specs=[
            pl.BlockSpec((1, 1), lambda i: (0, 0)),
            pl.BlockSpec((1, 2 * TOP_K, LENGTH, HB, HEAD_DIM, BATCH), _pt_map),
        ],
        out_shape=[
            jax.ShapeDtypeStruct((1, 1), jnp.float32),
            jax.ShapeDtypeStruct(
                (NUM_LAYERS, 2 * TOP_K, LENGTH, NUM_HEADS, HEAD_DIM, BATCH),
                jnp.float32),
        ],
        scratch_shapes=[pltpu.VMEM((POOL_SIZE, 8 * 64), jnp.float32)],
    )(ssum, inv_x, keyn, pt)

    a6 = a7.reshape(NUM_LAYERS, 2, TOP_K * LENGTH, NUM_HEADS, HEAD_DIM, BATCH)
    batched_prompt = jnp.transpose(a6, (0, 5, 1, 2, 3, 4))
    return (batched_prompt, rs[0, 0])
